# baseline (speedup 1.0000x reference)
import jax
import jax.numpy as jnp
from jax.experimental import pallas as pl
from jax.experimental.pallas import tpu as pltpu


def _emb_kernel(row_ref, col_ref, pose_ref, p_out_ref, m_out_ref, m_scratch):
    d = col_ref.shape[1]
    bb, _, hw = m_out_ref.shape
    h = 32
    w = hw // h

    @pl.when(pl.program_id(0) == 0)
    def _init():
        ct = col_ref[1 : w + 1, :].T
        rt = row_ref[1 : h + 1, :].T
        top = jnp.broadcast_to(ct[:, None, :], (d, h, w)).reshape(d, hw)
        bot = jnp.broadcast_to(rt[:, :, None], (d, h, w)).reshape(d, hw)
        m_scratch[...] = jnp.concatenate([top, bot], axis=0)

    m_out_ref[...] = jnp.broadcast_to(m_scratch[...][None], (bb, 2 * d, hw))

    pe = pose_ref[0, :]
    p2 = jnp.concatenate([pe, pe])
    p_out_ref[...] = jnp.broadcast_to(p2[None, :], (bb, 2 * d))


def kernel(x, row_embed, col_embed, pose_token_embed):
    B = x.shape[0]
    h, w = x.shape[-2], x.shape[-1]
    d = col_embed.shape[1]
    BB = 8
    grid = (B // BB,)

    p_emb, m_flat = pl.pallas_call(
        _emb_kernel,
        grid=grid,
        in_specs=[
            pl.BlockSpec(row_embed.shape, lambda b: (0, 0)),
            pl.BlockSpec(col_embed.shape, lambda b: (0, 0)),
            pl.BlockSpec(pose_token_embed.shape, lambda b: (0, 0)),
        ],
        out_specs=[
            pl.BlockSpec((BB, 2 * d), lambda b: (b, 0)),
            pl.BlockSpec((BB, 2 * d, h * w), lambda b: (b, 0, 0)),
        ],
        out_shape=[
            jax.ShapeDtypeStruct((B, 2 * d), jnp.float32),
            jax.ShapeDtypeStruct((B, 2 * d, h * w), jnp.float32),
        ],
        scratch_shapes=[pltpu.VMEM((2 * d, h * w), jnp.float32)],
    )(row_embed, col_embed, pose_token_embed)
    return (p_emb, m_flat.reshape(B, 2 * d, h, w))
